# Initial kernel scaffold; baseline (speedup 1.0000x reference)
import jax
import jax.numpy as jnp
from jax.experimental import pallas as pl


def kernel(x, seg, mask, tok_embed, W, b):
    raise NotImplementedError("write your pallas kernel here")


# speedup vs baseline: 2.2625x; 2.2625x over previous
import functools

import jax
import jax.numpy as jnp
from jax import lax
from jax.experimental import pallas as pl
from jax.experimental.pallas import tpu as pltpu
from jax.experimental.pallas import tpu_sc as plsc

B = 4096
L = 200
DIM = 32
N_LABELS = 10

NC = 2
NS = 16
NW = NC * NS
ROWS_PER_W = B // NW
CHUNK_ROWS = 8
CHUNK_TOKS = CHUNK_ROWS * L
N_CHUNKS = ROWS_PER_W // CHUNK_ROWS
GATHER_N = 128
N_FULL_G = CHUNK_TOKS // GATHER_N
TAIL_G = CHUNK_TOKS - N_FULL_G * GATHER_N


def _pool_body(x_hbm, table_hbm, out_hbm, idx_v, rows_v, acc_v, sem):
    wid = lax.axis_index("s") * NC + lax.axis_index("c")

    def chunk_body(c, _):
        row_base = wid * ROWS_PER_W + c * CHUNK_ROWS
        tok_base = row_base * L
        pltpu.sync_copy(x_hbm.at[pl.ds(tok_base, CHUNK_TOKS)], idx_v)
        copies = []
        for j in range(N_FULL_G):
            copies.append(pltpu.async_copy(
                table_hbm.at[idx_v.at[pl.ds(j * GATHER_N, GATHER_N)]],
                rows_v.at[pl.ds(j * GATHER_N, GATHER_N)],
                sem))
        if TAIL_G:
            copies.append(pltpu.async_copy(
                table_hbm.at[idx_v.at[pl.ds(N_FULL_G * GATHER_N, TAIL_G)]],
                rows_v.at[pl.ds(N_FULL_G * GATHER_N, TAIL_G)],
                sem))
        for cp in copies:
            cp.wait()

        def row_body(r, _):
            def tok_body(t, carry):
                a0, a1 = carry
                i = r * L + t
                return (a0 + rows_v[i, pl.ds(0, 16)],
                        a1 + rows_v[i, pl.ds(16, 16)])
            a0, a1 = lax.fori_loop(
                0, L, tok_body,
                (jnp.zeros((16,), jnp.float32), jnp.zeros((16,), jnp.float32)),
                unroll=8)
            acc_v[r, pl.ds(0, 16)] = a0
            acc_v[r, pl.ds(16, 16)] = a1
            return 0

        lax.fori_loop(0, CHUNK_ROWS, row_body, 0)
        pltpu.sync_copy(acc_v, out_hbm.at[pl.ds(row_base, CHUNK_ROWS)])
        return 0

    lax.fori_loop(0, N_CHUNKS, chunk_body, 0)


_pool = functools.partial(
    pl.kernel,
    mesh=plsc.VectorSubcoreMesh(core_axis_name="c", subcore_axis_name="s"),
    compiler_params=pltpu.CompilerParams(use_tc_tiling_on_sc=False),
    out_type=jax.ShapeDtypeStruct((B, DIM), jnp.float32),
    scratch_types=[
        pltpu.VMEM((CHUNK_TOKS,), jnp.int32),
        pltpu.VMEM((CHUNK_TOKS, DIM), jnp.float32),
        pltpu.VMEM((CHUNK_ROWS, DIM), jnp.float32),
        pltpu.SemaphoreType.DMA,
    ],
)(_pool_body)


def _clf_body(p_ref, w_ref, b_ref, o_ref):
    o_ref[...] = jnp.dot(p_ref[...], w_ref[...],
                         preferred_element_type=jnp.float32) + b_ref[...]


def _clf(pooled, W, b2):
    return pl.pallas_call(
        _clf_body,
        out_shape=jax.ShapeDtypeStruct((B, N_LABELS), jnp.float32),
    )(pooled, W, b2)


def kernel(x, seg, mask, tok_embed, W, b):
    x_flat = x.reshape(-1).astype(jnp.int32)
    pooled = _pool(x_flat, tok_embed)
    return _clf(pooled, W, b.reshape(1, N_LABELS))
